# baseline (speedup 1.0000x reference)
import jax
import jax.numpy as jnp
from jax.experimental import pallas as pl

D = 2048
L = 16384
K = 64
N = 8192


def _enc_kernel(x_ref, w_ref, b_ref, a_ref):
    pre = jax.lax.dot_general(
        x_ref[...], w_ref[...], (((1,), (1,)), ((), ())),
        preferred_element_type=jnp.float32)
    pre = pre + b_ref[...]
    a_ref[...] = pre * jax.nn.sigmoid(pre)


def _encode(x, w_enc, b_enc):
    TT, LB = 512, 2048
    grid = (L // LB, N // TT)
    return pl.pallas_call(
        _enc_kernel,
        grid=grid,
        in_specs=[
            pl.BlockSpec((TT, D), lambda l, t: (t, 0)),
            pl.BlockSpec((LB, D), lambda l, t: (l, 0)),
            pl.BlockSpec((1, LB), lambda l, t: (0, l)),
        ],
        out_specs=pl.BlockSpec((TT, LB), lambda l, t: (t, l)),
        out_shape=jax.ShapeDtypeStruct((N, L), jnp.float32),
    )(x, w_enc, b_enc.reshape(1, L))


NSL = 128
SD = L // NSL


def _select_kernel(a_ref, s_ref, v_ref, i_ref):
    TT = a_ref.shape[0]
    a = a_ref[...]
    a3 = a.reshape(TT, SD, NSL)
    siota = jax.lax.broadcasted_iota(jnp.int32, (TT, NSL), 1)
    jiota = jax.lax.broadcasted_iota(jnp.int32, (TT, SD), 1)
    col = jax.lax.broadcasted_iota(jnp.int32, (TT, K), 1)

    M = jnp.max(a3, axis=1)
    j3 = jax.lax.broadcasted_iota(jnp.int32, (TT, SD, NSL), 1)
    widx = jnp.min(jnp.where(a3 == M[:, None, :], j3, SD), axis=1)

    def body(k, carry):
        M, widx, vs, ids = carry
        v = jnp.max(M, axis=1, keepdims=True)
        cand = jnp.where(M == v, widx * NSL + siota, L)
        l_star = jnp.min(cand, axis=1, keepdims=True)
        s_star = jax.lax.rem(l_star, NSL)
        wi_star = jax.lax.div(l_star, NSL)
        vs = jnp.where(col == k, v, vs)
        ids = jnp.where(col == k, l_star, ids)
        gidx = jnp.broadcast_to(s_star[:, None, :], (TT, SD, 1))
        sv = jnp.take_along_axis(a3, gidx, axis=2,
                                 mode="promise_in_bounds")[:, :, 0]
        keep = (sv < v) | ((sv == v) & (jiota > wi_star))
        new_max = jnp.max(jnp.where(keep, sv, -jnp.inf), axis=1, keepdims=True)
        new_wi = jnp.min(jnp.where((sv == new_max) & keep, jiota, SD),
                         axis=1, keepdims=True)
        M = jnp.where(siota == s_star, new_max, M)
        widx = jnp.where(siota == s_star, new_wi, widx)
        return M, widx, vs, ids

    v0 = jnp.zeros((TT, K), jnp.float32)
    i0 = jnp.zeros((TT, K), jnp.int32)
    _, _, v, idx = jax.lax.fori_loop(0, K, body, (M, widx, v0, i0))
    cut = v[:, K - 1:K]
    s_ref[...] = jnp.where(a >= cut, a, 0.0)
    v_ref[...] = v
    i_ref[...] = idx


def _select(a):
    TT = 64
    grid = (N // TT,)
    return pl.pallas_call(
        _select_kernel,
        grid=grid,
        in_specs=[pl.BlockSpec((TT, L), lambda t: (t, 0))],
        out_specs=[
            pl.BlockSpec((TT, L), lambda t: (t, 0)),
            pl.BlockSpec((TT, K), lambda t: (t, 0)),
            pl.BlockSpec((TT, K), lambda t: (t, 0)),
        ],
        out_shape=[
            jax.ShapeDtypeStruct((N, L), jnp.float32),
            jax.ShapeDtypeStruct((N, K), jnp.float32),
            jax.ShapeDtypeStruct((N, K), jnp.int32),
        ],
    )(a)


def _dec_kernel(s_ref, w_ref, o_ref):
    k = pl.program_id(1)
    part = jax.lax.dot_general(
        s_ref[...].astype(jnp.bfloat16), w_ref[...],
        (((1,), (1,)), ((), ())),
        preferred_element_type=jnp.float32)

    @pl.when(k == 0)
    def _():
        o_ref[...] = part

    @pl.when(k != 0)
    def _():
        o_ref[...] += part


def _decode(s, w_dec):
    TT, KB = 512, 2048
    grid = (N // TT, L // KB)
    return pl.pallas_call(
        _dec_kernel,
        grid=grid,
        in_specs=[
            pl.BlockSpec((TT, KB), lambda t, k: (t, k)),
            pl.BlockSpec((D, KB), lambda t, k: (0, k)),
        ],
        out_specs=pl.BlockSpec((TT, D), lambda t, k: (t, 0)),
        out_shape=jax.ShapeDtypeStruct((N, D), jnp.float32),
    )(s, w_dec)


def kernel(inputs, W_enc, b_enc, W_dec):
    x_bf = inputs.astype(jnp.bfloat16)
    we_bf = W_enc.astype(jnp.bfloat16)
    wd_bf = W_dec.astype(jnp.bfloat16)
    a = _encode(x_bf, we_bf, b_enc)
    sparse_latents, active_values, active_indices = _select(a)
    delta_hat = _decode(sparse_latents, wd_bf)
    return (delta_hat, sparse_latents, active_indices, active_values)


# speedup vs baseline: 4.0278x; 4.0278x over previous
import jax
import jax.numpy as jnp
from jax.experimental import pallas as pl

D = 2048
L = 16384
K = 64
N = 8192


def _enc_kernel(x_ref, w_ref, b_ref, a_ref):
    pre = jax.lax.dot_general(
        x_ref[...], w_ref[...], (((1,), (1,)), ((), ())),
        preferred_element_type=jnp.float32)
    pre = pre + b_ref[...]
    a_ref[...] = pre * jax.nn.sigmoid(pre)


def _encode(x, w_enc, b_enc):
    TT, LB = 512, 2048
    grid = (L // LB, N // TT)
    return pl.pallas_call(
        _enc_kernel,
        grid=grid,
        in_specs=[
            pl.BlockSpec((TT, D), lambda l, t: (t, 0)),
            pl.BlockSpec((LB, D), lambda l, t: (l, 0)),
            pl.BlockSpec((1, LB), lambda l, t: (0, l)),
        ],
        out_specs=pl.BlockSpec((TT, LB), lambda l, t: (t, l)),
        out_shape=jax.ShapeDtypeStruct((N, L), jnp.float32),
    )(x, w_enc, b_enc.reshape(1, L))


def _select_kernel(a_ref, s_ref, v_ref, i_ref, scratch_ref):
    TT = a_ref.shape[0]
    a = a_ref[...]
    scratch_ref[...] = a
    iota = jax.lax.broadcasted_iota(jnp.int32, (TT, L), 1)
    col = jax.lax.broadcasted_iota(jnp.int32, (TT, K), 1)

    def body(k, carry):
        vs, ids = carry
        cur = scratch_ref[...]
        m = jnp.max(cur, axis=1, keepdims=True)
        hit = cur == m
        idx = jnp.min(jnp.where(hit, iota, L), axis=1, keepdims=True)
        scratch_ref[...] = jnp.where(iota == idx, -jnp.inf, cur)
        vs = jnp.where(col == k, m, vs)
        ids = jnp.where(col == k, idx, ids)
        return vs, ids

    v0 = jnp.zeros((TT, K), jnp.float32)
    i0 = jnp.zeros((TT, K), jnp.int32)
    v, idx = jax.lax.fori_loop(0, K, body, (v0, i0))
    cut = v[:, K - 1:K]
    s_ref[...] = jnp.where(a >= cut, a, 0.0)
    v_ref[...] = v
    i_ref[...] = idx


def _select(a):
    from jax.experimental.pallas import tpu as pltpu
    TT = 128
    grid = (N // TT,)
    return pl.pallas_call(
        _select_kernel,
        grid=grid,
        in_specs=[pl.BlockSpec((TT, L), lambda t: (t, 0))],
        scratch_shapes=[pltpu.VMEM((TT, L), jnp.float32)],
        out_specs=[
            pl.BlockSpec((TT, L), lambda t: (t, 0)),
            pl.BlockSpec((TT, K), lambda t: (t, 0)),
            pl.BlockSpec((TT, K), lambda t: (t, 0)),
        ],
        out_shape=[
            jax.ShapeDtypeStruct((N, L), jnp.float32),
            jax.ShapeDtypeStruct((N, K), jnp.float32),
            jax.ShapeDtypeStruct((N, K), jnp.int32),
        ],
    )(a)


def _dec_kernel(s_ref, w_ref, o_ref):
    k = pl.program_id(1)
    part = jax.lax.dot_general(
        s_ref[...].astype(jnp.bfloat16), w_ref[...],
        (((1,), (1,)), ((), ())),
        preferred_element_type=jnp.float32)

    @pl.when(k == 0)
    def _():
        o_ref[...] = part

    @pl.when(k != 0)
    def _():
        o_ref[...] += part


def _decode(s, w_dec):
    TT, KB = 512, 2048
    grid = (N // TT, L // KB)
    return pl.pallas_call(
        _dec_kernel,
        grid=grid,
        in_specs=[
            pl.BlockSpec((TT, KB), lambda t, k: (t, k)),
            pl.BlockSpec((D, KB), lambda t, k: (0, k)),
        ],
        out_specs=pl.BlockSpec((TT, D), lambda t, k: (t, 0)),
        out_shape=jax.ShapeDtypeStruct((N, D), jnp.float32),
    )(s, w_dec)


def kernel(inputs, W_enc, b_enc, W_dec):
    x_bf = inputs.astype(jnp.bfloat16)
    we_bf = W_enc.astype(jnp.bfloat16)
    wd_bf = W_dec.astype(jnp.bfloat16)
    a = _encode(x_bf, we_bf, b_enc)
    sparse_latents, active_values, active_indices = _select(a)
    delta_hat = _decode(sparse_latents, wd_bf)
    return (delta_hat, sparse_latents, active_indices, active_values)
